# Initial kernel scaffold; baseline (speedup 1.0000x reference)
import jax
import jax.numpy as jnp
from jax.experimental import pallas as pl


def kernel(input_tensor, token_type_ids, token_type_table):
    raise NotImplementedError("write your pallas kernel here")


# speedup vs baseline: 1.9532x; 1.9532x over previous
import functools

import jax
import jax.numpy as jnp
from jax import lax
from jax.experimental import pallas as pl
from jax.experimental.pallas import tpu as pltpu

_VOCAB = 16
_BLK = 1024


def _body(ids_ref, x_ref, tbl_ref, o_ref):
    ids = ids_ref[0, 0, :]
    onehot = (ids[:, None] == lax.broadcasted_iota(jnp.int32, (_BLK, _VOCAB), 1)
              ).astype(jnp.float32)
    emb = jnp.dot(onehot, tbl_ref[:, :], preferred_element_type=jnp.float32)
    o_ref[:, :] = x_ref[:, :] + emb


def kernel(input_tensor, token_type_ids, token_type_table):
    b, s, e = input_tensor.shape
    n = b * s
    nb = n // _BLK
    x = input_tensor.reshape(n, e)
    ids3 = token_type_ids.reshape(nb, 1, _BLK).astype(jnp.int32)

    out = pl.pallas_call(
        _body,
        grid=(nb,),
        in_specs=[
            pl.BlockSpec((1, 1, _BLK), lambda i: (i, 0, 0)),
            pl.BlockSpec((_BLK, e), lambda i: (i, 0)),
            pl.BlockSpec((_VOCAB, e), lambda i: (0, 0)),
        ],
        out_specs=pl.BlockSpec((_BLK, e), lambda i: (i, 0)),
        out_shape=jax.ShapeDtypeStruct((n, e), jnp.float32),
        compiler_params=pltpu.CompilerParams(
            dimension_semantics=("arbitrary",),
        ),
    )(ids3, x, token_type_table)
    return out.reshape(b, s, e)
